# baseline (speedup 1.0000x reference)
import jax
import jax.numpy as jnp
from jax.experimental import pallas as pl
from jax.experimental.pallas import tpu as pltpu

_SWIGLU_SCALE = 1.702
_FB = 512
_NBUF = 4


def _copies(g_hbm, u_hbm, d_hbm, gbuf, ubuf, dbuf, sems, i, nf, slot):
    e = i // nf
    fs = jax.lax.rem(i, nf) * _FB
    return (
        pltpu.make_async_copy(g_hbm.at[e, :, pl.ds(fs, _FB)], gbuf.at[slot],
                              sems.at[slot, 0]),
        pltpu.make_async_copy(u_hbm.at[e, :, pl.ds(fs, _FB)], ubuf.at[slot],
                              sems.at[slot, 1]),
        pltpu.make_async_copy(d_hbm.at[e, pl.ds(fs, _FB), :], dbuf.at[slot],
                              sems.at[slot, 2]),
    )


def _moe_body(x_ref, rw_ref, g_hbm, u_hbm, d_hbm, out_ref,
              gbuf, ubuf, dbuf, cw_ref, sems):
    x = x_ref[...]
    n_exp = g_hbm.shape[0]
    nf = g_hbm.shape[2] // _FB
    nc = n_exp * nf

    for i in range(_NBUF - 1):
        for c in _copies(g_hbm, u_hbm, d_hbm, gbuf, ubuf, dbuf, sems,
                         i, nf, i):
            c.start()

    logits = jnp.dot(x, rw_ref[...], preferred_element_type=jnp.float32)
    idx = jax.lax.broadcasted_iota(jnp.int32, logits.shape, 1)
    m1 = jnp.max(logits, axis=-1, keepdims=True)
    i1 = jnp.min(jnp.where(logits == m1, idx, n_exp), axis=-1, keepdims=True)
    l2 = jnp.where(idx == i1, -jnp.inf, logits)
    m2 = jnp.max(l2, axis=-1, keepdims=True)
    i2 = jnp.min(jnp.where(l2 == m2, idx, n_exp), axis=-1, keepdims=True)
    top2 = (idx == i1) | (idx == i2)
    w = jnp.where(top2, jnp.exp(logits - m1), 0.0)
    cw_ref[...] = w / jnp.sum(w, axis=-1, keepdims=True)

    out_ref[...] = jnp.zeros_like(out_ref)
    xb = x.astype(jnp.bfloat16)

    def step(i, _):
        slot = jax.lax.rem(i, _NBUF)
        nxt = i + _NBUF - 1

        @pl.when(nxt < nc)
        def _():
            for c in _copies(g_hbm, u_hbm, d_hbm, gbuf, ubuf, dbuf, sems,
                             nxt, nf, jax.lax.rem(nxt, _NBUF)):
                c.start()

        cg, cu, cd = _copies(g_hbm, u_hbm, d_hbm, gbuf, ubuf, dbuf, sems,
                             i, nf, slot)
        e = i // nf
        lane = jax.lax.broadcasted_iota(jnp.int32, cw_ref.shape, 1)
        w_e = jnp.sum(jnp.where(lane == e, cw_ref[...], 0.0),
                      axis=-1, keepdims=True)
        cg.wait()
        g = jnp.dot(xb, gbuf[slot].astype(jnp.bfloat16),
                    preferred_element_type=jnp.float32)
        cu.wait()
        u = jnp.dot(xb, ubuf[slot].astype(jnp.bfloat16),
                    preferred_element_type=jnp.float32)
        act = g * jax.nn.sigmoid(_SWIGLU_SCALE * g) * u
        cd.wait()
        out_ref[...] += jnp.dot((act * w_e).astype(jnp.bfloat16),
                                dbuf[slot].astype(jnp.bfloat16),
                                preferred_element_type=jnp.float32)
        return ()

    jax.lax.fori_loop(0, nc, step, ())


def kernel(hidden_states, router_weight, gate_proj, up_proj, down_proj):
    b, s, h = hidden_states.shape
    e, _, f = gate_proj.shape
    t = b * s
    x = hidden_states.reshape(t, h)

    out = pl.pallas_call(
        _moe_body,
        in_specs=[
            pl.BlockSpec(memory_space=pltpu.MemorySpace.VMEM),
            pl.BlockSpec(memory_space=pltpu.MemorySpace.VMEM),
            pl.BlockSpec(memory_space=pltpu.MemorySpace.HBM),
            pl.BlockSpec(memory_space=pltpu.MemorySpace.HBM),
            pl.BlockSpec(memory_space=pltpu.MemorySpace.HBM),
        ],
        out_specs=pl.BlockSpec(memory_space=pltpu.MemorySpace.VMEM),
        out_shape=jax.ShapeDtypeStruct((t, h), jnp.float32),
        scratch_shapes=[
            pltpu.VMEM((_NBUF, h, _FB), jnp.float32),
            pltpu.VMEM((_NBUF, h, _FB), jnp.float32),
            pltpu.VMEM((_NBUF, _FB, h), jnp.float32),
            pltpu.VMEM((t, e), jnp.float32),
            pltpu.SemaphoreType.DMA((_NBUF, 3)),
        ],
        compiler_params=pltpu.CompilerParams(
            vmem_limit_bytes=63 * 1024 * 1024,
        ),
    )(x, router_weight, gate_proj, up_proj, down_proj)
    return out.reshape(b, s, h)


# speedup vs baseline: 1.0345x; 1.0345x over previous
import jax
import jax.numpy as jnp
from jax.experimental import pallas as pl
from jax.experimental.pallas import tpu as pltpu

_SWIGLU_SCALE = 1.702
_FB = 512


def _moe_body(x_ref, rw_ref, gate_ref, up_ref, down_ref, out_ref, cw_ref):
    e = pl.program_id(0)
    f = pl.program_id(1)
    x = x_ref[...]

    @pl.when((e == 0) & (f == 0))
    def _router():
        logits = jnp.dot(x, rw_ref[...], preferred_element_type=jnp.float32)
        n_e = logits.shape[-1]
        idx = jax.lax.broadcasted_iota(jnp.int32, logits.shape, 1)
        m1 = jnp.max(logits, axis=-1, keepdims=True)
        i1 = jnp.min(jnp.where(logits == m1, idx, n_e), axis=-1, keepdims=True)
        l2 = jnp.where(idx == i1, -jnp.inf, logits)
        m2 = jnp.max(l2, axis=-1, keepdims=True)
        i2 = jnp.min(jnp.where(l2 == m2, idx, n_e), axis=-1, keepdims=True)
        top2 = (idx == i1) | (idx == i2)
        w = jnp.where(top2, jnp.exp(logits - m1), 0.0)
        cw_ref[...] = w / jnp.sum(w, axis=-1, keepdims=True)
        out_ref[...] = jnp.zeros_like(out_ref)

    xb = x.astype(jnp.bfloat16)
    g = jnp.dot(xb, gate_ref[0].astype(jnp.bfloat16),
                preferred_element_type=jnp.float32)
    u = jnp.dot(xb, up_ref[0].astype(jnp.bfloat16),
                preferred_element_type=jnp.float32)
    act = g * jax.nn.sigmoid(_SWIGLU_SCALE * g) * u
    lane = jax.lax.broadcasted_iota(jnp.int32, cw_ref.shape, 1)
    w_e = jnp.sum(jnp.where(lane == e, cw_ref[...], 0.0), axis=-1, keepdims=True)
    out_ref[...] += jnp.dot((act * w_e).astype(jnp.bfloat16),
                            down_ref[0].astype(jnp.bfloat16),
                            preferred_element_type=jnp.float32)


def kernel(hidden_states, router_weight, gate_proj, up_proj, down_proj):
    b, s, h = hidden_states.shape
    e, _, f = gate_proj.shape
    t = b * s
    x = hidden_states.reshape(t, h)
    nf = f // _FB

    out = pl.pallas_call(
        _moe_body,
        grid=(e, nf),
        in_specs=[
            pl.BlockSpec((t, h), lambda ei, fi: (0, 0)),
            pl.BlockSpec((h, e), lambda ei, fi: (0, 0)),
            pl.BlockSpec((1, h, _FB), lambda ei, fi: (ei, 0, fi)),
            pl.BlockSpec((1, h, _FB), lambda ei, fi: (ei, 0, fi)),
            pl.BlockSpec((1, _FB, h), lambda ei, fi: (ei, fi, 0)),
        ],
        out_specs=pl.BlockSpec((t, h), lambda ei, fi: (0, 0)),
        out_shape=jax.ShapeDtypeStruct((t, h), jnp.float32),
        scratch_shapes=[pltpu.VMEM((t, e), jnp.float32)],
        compiler_params=pltpu.CompilerParams(
            dimension_semantics=("arbitrary", "arbitrary"),
        ),
    )(x, router_weight, gate_proj, up_proj, down_proj)
    return out.reshape(b, s, h)


# speedup vs baseline: 1.1018x; 1.0651x over previous
import jax
import jax.numpy as jnp
from jax.experimental import pallas as pl
from jax.experimental.pallas import tpu as pltpu

_SWIGLU_SCALE = 1.702
_FB = 512


def _moe_body(x_ref, rw_ref, gate_ref, up_ref, down_ref, out_ref, cw_ref):
    e = pl.program_id(0)
    f = pl.program_id(1)
    x = x_ref[:, 0, :]

    @pl.when((e == 0) & (f == 0))
    def _router():
        logits = jnp.dot(x, rw_ref[...], preferred_element_type=jnp.float32)
        n_e = logits.shape[-1]
        idx = jax.lax.broadcasted_iota(jnp.int32, logits.shape, 1)
        m1 = jnp.max(logits, axis=-1, keepdims=True)
        i1 = jnp.min(jnp.where(logits == m1, idx, n_e), axis=-1, keepdims=True)
        l2 = jnp.where(idx == i1, -jnp.inf, logits)
        m2 = jnp.max(l2, axis=-1, keepdims=True)
        i2 = jnp.min(jnp.where(l2 == m2, idx, n_e), axis=-1, keepdims=True)
        top2 = (idx == i1) | (idx == i2)
        w = jnp.where(top2, jnp.exp(logits - m1), 0.0)
        cw_ref[...] = w / jnp.sum(w, axis=-1, keepdims=True)
        out_ref[:, 0, :] = jnp.zeros(out_ref.shape[::2], out_ref.dtype)

    xb = x.astype(jnp.bfloat16)
    g = jnp.dot(xb, gate_ref[0].astype(jnp.bfloat16),
                preferred_element_type=jnp.float32)
    u = jnp.dot(xb, up_ref[0].astype(jnp.bfloat16),
                preferred_element_type=jnp.float32)
    act = g * jax.nn.sigmoid(_SWIGLU_SCALE * g) * u
    lane = jax.lax.broadcasted_iota(jnp.int32, cw_ref.shape, 1)
    w_e = jnp.sum(jnp.where(lane == e, cw_ref[...], 0.0), axis=-1, keepdims=True)
    out_ref[:, 0, :] += jnp.dot((act * w_e).astype(jnp.bfloat16),
                                down_ref[0].astype(jnp.bfloat16),
                                preferred_element_type=jnp.float32)


def kernel(hidden_states, router_weight, gate_proj, up_proj, down_proj):
    b, s, h = hidden_states.shape
    e, _, f = gate_proj.shape
    nf = f // _FB

    return pl.pallas_call(
        _moe_body,
        grid=(e, nf),
        in_specs=[
            pl.BlockSpec((b, s, h), lambda ei, fi: (0, 0, 0)),
            pl.BlockSpec((h, e), lambda ei, fi: (0, 0)),
            pl.BlockSpec((1, h, _FB), lambda ei, fi: (ei, 0, fi)),
            pl.BlockSpec((1, h, _FB), lambda ei, fi: (ei, 0, fi)),
            pl.BlockSpec((1, _FB, h), lambda ei, fi: (ei, fi, 0)),
        ],
        out_specs=pl.BlockSpec((b, s, h), lambda ei, fi: (0, 0, 0)),
        out_shape=jax.ShapeDtypeStruct((b, s, h), jnp.float32),
        scratch_shapes=[pltpu.VMEM((b * s, e), jnp.float32)],
        compiler_params=pltpu.CompilerParams(
            dimension_semantics=("arbitrary", "arbitrary"),
        ),
    )(hidden_states, router_weight, gate_proj, up_proj, down_proj)


# speedup vs baseline: 1.1537x; 1.0471x over previous
import jax
import jax.numpy as jnp
from jax.experimental import pallas as pl
from jax.experimental.pallas import tpu as pltpu

_SWIGLU_SCALE = 1.702
_FB = 512


def _moe_body(x_ref, rw_ref, gate_ref, up_ref, down_ref, out_ref, cw_ref):
    e = pl.program_id(0)
    f = pl.program_id(1)
    x = x_ref[:, 0, :]

    @pl.when((e == 0) & (f == 0))
    def _router():
        logits = jax.lax.dot_general(
            x, rw_ref[...], (((1,), (1,)), ((), ())),
            preferred_element_type=jnp.float32)
        n_e = logits.shape[-1]
        idx = jax.lax.broadcasted_iota(jnp.int32, logits.shape, 1)
        m1 = jnp.max(logits, axis=-1, keepdims=True)
        i1 = jnp.min(jnp.where(logits == m1, idx, n_e), axis=-1, keepdims=True)
        l2 = jnp.where(idx == i1, -jnp.inf, logits)
        m2 = jnp.max(l2, axis=-1, keepdims=True)
        i2 = jnp.min(jnp.where(l2 == m2, idx, n_e), axis=-1, keepdims=True)
        top2 = (idx == i1) | (idx == i2)
        w = jnp.where(top2, jnp.exp(logits - m1), 0.0)
        cw_ref[...] = w / jnp.sum(w, axis=-1, keepdims=True)
        out_ref[:, 0, :] = jnp.zeros(out_ref.shape[::2], out_ref.dtype)

    xb = x.astype(jnp.bfloat16)
    g = jnp.dot(xb, gate_ref[0].astype(jnp.bfloat16),
                preferred_element_type=jnp.float32)
    u = jnp.dot(xb, up_ref[0].astype(jnp.bfloat16),
                preferred_element_type=jnp.float32)
    act = g * jax.nn.sigmoid(_SWIGLU_SCALE * g) * u
    lane = jax.lax.broadcasted_iota(jnp.int32, cw_ref.shape, 1)
    w_e = jnp.sum(jnp.where(lane == e, cw_ref[...], 0.0), axis=-1, keepdims=True)
    out_ref[:, 0, :] += jnp.dot((act * w_e).astype(jnp.bfloat16),
                                down_ref[0].astype(jnp.bfloat16),
                                preferred_element_type=jnp.float32)


def kernel(hidden_states, router_weight, gate_proj, up_proj, down_proj):
    b, s, h = hidden_states.shape
    e, _, f = gate_proj.shape
    nf = f // _FB

    return pl.pallas_call(
        _moe_body,
        grid=(e, nf),
        in_specs=[
            pl.BlockSpec((b, s, h), lambda ei, fi: (0, 0, 0)),
            pl.BlockSpec((e, h), lambda ei, fi: (0, 0)),
            pl.BlockSpec((1, h, _FB), lambda ei, fi: (ei, 0, fi)),
            pl.BlockSpec((1, h, _FB), lambda ei, fi: (ei, 0, fi)),
            pl.BlockSpec((1, _FB, h), lambda ei, fi: (ei, fi, 0)),
        ],
        out_specs=pl.BlockSpec((b, s, h), lambda ei, fi: (0, 0, 0)),
        out_shape=jax.ShapeDtypeStruct((b, s, h), jnp.float32),
        scratch_shapes=[pltpu.VMEM((b * s, e), jnp.float32)],
        compiler_params=pltpu.CompilerParams(
            dimension_semantics=("arbitrary", "arbitrary"),
        ),
    )(hidden_states, router_weight.T, gate_proj, up_proj, down_proj)
